# Initial kernel scaffold; baseline (speedup 1.0000x reference)
import jax
import jax.numpy as jnp
from jax.experimental import pallas as pl


def kernel(indices, table):
    raise NotImplementedError("write your pallas kernel here")


# speedup vs baseline: 3.2316x; 3.2316x over previous
import functools

import jax
import jax.numpy as jnp
from jax import lax
from jax.experimental import pallas as pl
from jax.experimental.pallas import tpu as pltpu
from jax.experimental.pallas import tpu_sc as plsc

NUM_CORES = 2
NUM_SUBCORES = 16
NW = NUM_CORES * NUM_SUBCORES
G = 128


@functools.partial(jax.jit, static_argnums=(2, 3))
def _sc_gather(flat_idx, table, n_rows, d):
    per_w = n_rows // NW
    ng = per_w // G
    mesh = plsc.VectorSubcoreMesh(
        core_axis_name="c",
        subcore_axis_name="s",
        num_cores=NUM_CORES,
        num_subcores=NUM_SUBCORES,
    )

    @functools.partial(
        pl.kernel,
        out_type=jax.ShapeDtypeStruct((n_rows, d), jnp.float32),
        mesh=mesh,
        scratch_types=[
            pltpu.VMEM((G,), jnp.int32),
            pltpu.VMEM((G, d), jnp.float32),
            pltpu.SemaphoreType.DMA,
        ],
    )
    def k(idx_hbm, table_hbm, out_hbm, idx_v, rows_v, gsem):
        wid = lax.axis_index("s") * NUM_CORES + lax.axis_index("c")
        base = wid * per_w

        def body(g, carry):
            off = base + g * G
            pltpu.sync_copy(idx_hbm.at[pl.ds(off, G)], idx_v)
            pltpu.async_copy(table_hbm.at[idx_v], rows_v, gsem).wait()
            pltpu.sync_copy(rows_v, out_hbm.at[pl.ds(off, G)])
            return carry

        lax.fori_loop(0, ng, body, 0)

    return k(flat_idx, table)


def kernel(indices, table):
    b, f = indices.shape
    d = table.shape[1]
    flat = indices.reshape(b * f).astype(jnp.int32)
    out = _sc_gather(flat, table, b * f, d)
    return out.reshape(b, f, d)


# speedup vs baseline: 4.0611x; 1.2567x over previous
import functools

import jax
import jax.numpy as jnp
from jax import lax
from jax.experimental import pallas as pl
from jax.experimental.pallas import tpu as pltpu
from jax.experimental.pallas import tpu_sc as plsc

NUM_CORES = 2
NUM_SUBCORES = 16
NW = NUM_CORES * NUM_SUBCORES
G = 128
NBUF = 4


@functools.partial(jax.jit, static_argnums=(2, 3))
def _sc_gather(flat_idx, table, n_rows, d):
    per_w = n_rows // NW
    ng = per_w // G
    nout = ng // NBUF
    mesh = plsc.VectorSubcoreMesh(
        core_axis_name="c",
        subcore_axis_name="s",
        num_cores=NUM_CORES,
        num_subcores=NUM_SUBCORES,
    )

    @functools.partial(
        pl.kernel,
        out_type=jax.ShapeDtypeStruct((n_rows, d), jnp.float32),
        mesh=mesh,
        scratch_types=[
            pltpu.VMEM((NBUF, G), jnp.int32),
            pltpu.VMEM((NBUF, G, d), jnp.float32),
        ]
        + [pltpu.SemaphoreType.DMA] * (2 * NBUF),
    )
    def k(idx_hbm, table_hbm, out_hbm, idx_v, rows_v, *sems):
        gsems = sems[:NBUF]
        osems = sems[NBUF:]
        wid = lax.axis_index("s") * NUM_CORES + lax.axis_index("c")
        base = wid * per_w

        def idx_load(g, b):
            pltpu.sync_copy(idx_hbm.at[pl.ds(base + g * G, G)], idx_v.at[b])

        def gather_start(b):
            pltpu.async_copy(table_hbm.at[idx_v.at[b]], rows_v.at[b], gsems[b])

        def gather_wait(b):
            pltpu.make_async_copy(
                table_hbm.at[pl.ds(0, G)], rows_v.at[b], gsems[b]
            ).wait()

        def store_start(g, b):
            pltpu.async_copy(rows_v.at[b], out_hbm.at[pl.ds(base + g * G, G)], osems[b])

        def store_wait(b):
            pltpu.make_async_copy(
                rows_v.at[b], out_hbm.at[pl.ds(0, G)], osems[b]
            ).wait()

        for b in range(NBUF):
            idx_load(b, b)
            gather_start(b)

        def round_body(r, carry):
            for b in range(NBUF):
                gather_wait(b)
                store_start(r * NBUF + b, b)

            @pl.when(r < nout - 1)
            def _prefetch():
                for b in range(NBUF):
                    store_wait(b)
                    idx_load((r + 1) * NBUF + b, b)
                    gather_start(b)

            return carry

        lax.fori_loop(0, nout, round_body, 0)
        for b in range(NBUF):
            store_wait(b)

    return k(flat_idx, table)


def kernel(indices, table):
    b, f = indices.shape
    d = table.shape[1]
    flat = indices.reshape(b * f).astype(jnp.int32)
    out = _sc_gather(flat, table, b * f, d)
    return out.reshape(b, f, d)
